# baseline (speedup 1.0000x reference)
import functools
import math

import jax
import jax.numpy as jnp
from jax.experimental import pallas as pl
from jax.experimental.pallas import tpu as pltpu
from jax.experimental.pallas import tpu_sc as plsc

B = 1024
D = 64
N = 100000
K = 256
NPAD = 102400
CHUNK = 4096
NCHUNK = NPAD // CHUNK
G = 16
NG = NPAD // G
GPC = CHUNK // G
RB = 128
NRB = B // RB
RB3 = 32
NRB3 = B // RB3
NBLK = NPAD // 128
MARGIN = 0.1
SOFTMAX_TEMPERATURE = max(1.0, math.log(0.2 * K) / 40)

_NEG = -1e30


def _lane_iota(shape, axis):
    return jax.lax.broadcasted_iota(jnp.int32, shape, axis)


def _cmp_exchange(vals, j, want_max):
    key = vals[0]
    up = (_lane_iota(key.shape, key.ndim - 1) & j) == 0
    partners = tuple(
        jnp.where(up, jnp.roll(v, -j, axis=-1), jnp.roll(v, j, axis=-1))
        for v in vals
    )
    pkey = partners[0]
    take_self = jnp.logical_xor(want_max, key < pkey)
    return tuple(
        jnp.where(take_self, v, pv) for v, pv in zip(vals, partners)
    )


def _bitonic_sort(vals, n, desc):
    key = vals[0]
    lane = _lane_iota(key.shape, key.ndim - 1)
    k = 2
    while k <= n:
        blk = (lane & k) == 0 if k < n else jnp.full(key.shape, True)
        xbd = jnp.logical_xor(blk, desc)
        j = k // 2
        while j >= 1:
            up = (lane & j) == 0
            want_max = jnp.logical_xor(up, xbd)
            vals = _cmp_exchange(vals, j, want_max)
            j //= 2
        k *= 2
    return vals


def _bitonic_clean(vals, n, desc):
    key = vals[0]
    lane = _lane_iota(key.shape, key.ndim - 1)
    j = n // 2
    while j >= 1:
        up = (lane & j) == 0
        want_max = jnp.logical_not(jnp.logical_xor(up, desc))
        vals = _cmp_exchange(vals, j, want_max)
        j //= 2
    return vals


def _viota(shape):
    nd = len(shape)
    return _lane_iota(shape, nd - 2) * 128 + _lane_iota(shape, nd - 1)


def _vcmpx(vals, d, want_max):
    key = vals[0]
    if d < 128:
        up = (_lane_iota(key.shape, key.ndim - 1) & d) == 0
        partners = tuple(
            jnp.where(up, jnp.roll(v, -d, axis=-1), jnp.roll(v, d, axis=-1))
            for v in vals
        )
    else:
        d2 = d // 128
        a_sz = key.shape[-2]

        def xorax(v):
            blocks = []
            for base in range(0, a_sz, 2 * d2):
                blocks.append(v[..., base + d2:base + 2 * d2, :])
                blocks.append(v[..., base:base + d2, :])
            return jnp.concatenate(blocks, axis=-2)

        partners = tuple(xorax(v) for v in vals)
    pkey = partners[0]
    take_self = jnp.logical_xor(want_max, key < pkey)
    return tuple(
        jnp.where(take_self, v, pv) for v, pv in zip(vals, partners)
    )


def _vsort(vals, n, desc):
    key = vals[0]
    vio = _viota(key.shape)
    k = 2
    while k <= n:
        blk = (vio & k) == 0 if k < n else jnp.full(key.shape, True)
        xbd = jnp.logical_xor(blk, desc)
        j = k // 2
        while j >= 1:
            up = (vio & j) == 0
            want_max = jnp.logical_xor(up, xbd)
            vals = _vcmpx(vals, j, want_max)
            j //= 2
        k *= 2
    return vals


def _vclean(vals, n, desc):
    key = vals[0]
    vio = _viota(key.shape)
    j = n // 2
    while j >= 1:
        up = (vio & j) == 0
        want_max = jnp.logical_not(jnp.logical_xor(up, desc))
        vals = _vcmpx(vals, j, want_max)
        j //= 2
    return vals


def _k1_body(x_ref, k_ref, s_out, gidx_out, q_s, t_s):
    j = pl.program_id(1)

    @pl.when(j == 0)
    def _():
        xb = x_ref[...]
        nrm = jnp.sqrt(jnp.sum(xb * xb, axis=1, keepdims=True))
        q_s[...] = xb / jnp.maximum(nrm, 1e-12)
        t_s[...] = jnp.full((RB, K), -1, jnp.int32)

    q = q_s[...]
    s = jax.lax.dot_general(
        q, k_ref[...], (((1,), (1,)), ((), ())),
        preferred_element_type=jnp.float32,
    )
    col = _lane_iota((RB, CHUNK), 1) + j * CHUNK
    s = jnp.where(col < N, s, _NEG)
    s_out[...] = s

    mf = s
    for sh in (1, 2, 4, 8):
        mf = jnp.maximum(mf, jnp.roll(mf, -sh, axis=1))
    lsel = _lane_iota((CHUNK, GPC), 0) == _lane_iota((CHUNK, GPC), 1) * G
    psel = lsel.astype(jnp.float32)
    m = jax.lax.dot_general(
        mf, psel, (((1,), (0,)), ((), ())),
        preferred_element_type=jnp.float32,
    )
    u = jnp.clip((m + 1.0) * 131072.0, 0.0, 262143.0).astype(jnp.int32)
    g = _lane_iota((RB, GPC), 1) + j * GPC
    packed = (u << 13) | g

    pm = packed.reshape(RB, 2, 128)
    (ps,) = _vsort((pm,), GPC, False)
    t = t_s[...].reshape(RB, 2, 128)
    z = jnp.maximum(t, ps)
    (t,) = _vclean((z,), K, True)
    t2 = t.reshape(RB, K)
    t_s[...] = t2

    @pl.when(j == NCHUNK - 1)
    def _():
        gidx_out[...] = t2 & 0x1FFF


def _k3_body(sb_ref, vb_ref, gx_ref, y_ref, yhat_out, sm_out, loss_out, acc_s):
    i = pl.program_id(0)
    sb = sb_ref[...].reshape(RB3, K, 128)
    vb = vb_ref[...].reshape(RB3, K, 128)
    sbt = jnp.swapaxes(sb, 1, 2)
    vbt = jnp.swapaxes(vb, 1, 2)
    off = (gx_ref[...] & 7)[:, None, :]

    cs = sbt[:, 0:G, :]
    cv = vbt[:, 0:G, :]
    for o in range(1, 8):
        sel = off == o
        cs = jnp.where(sel, sbt[:, o * G:(o + 1) * G, :], cs)
        cv = jnp.where(sel, vbt[:, o * G:(o + 1) * G, :], cv)

    seg = 16
    cs = cs.reshape(RB3, seg, 2, 128)
    cv = cv.reshape(RB3, seg, 2, 128)
    desc0 = (_lane_iota((1, seg, 1, 1), 1) < seg // 2)
    cs, cv = _vsort((cs, cv), K, desc0)
    while seg > 1:
        half = seg // 2
        a_s, b_s = cs[:, :half], cs[:, half:]
        a_v, b_v = cv[:, :half], cv[:, half:]
        take_a = a_s >= b_s
        zs = jnp.where(take_a, a_s, b_s)
        zv = jnp.where(take_a, a_v, b_v)
        if half > 1:
            dirn = (_lane_iota((1, half, 1, 1), 1) < half // 2)
        else:
            dirn = True
        cs, cv = _vclean((zs, zv), K, dirn)
        seg = half
    cos = cs.reshape(RB3, K)
    vals = cv.reshape(RB3, K)

    yhat_out[...] = vals[:, 0:1]

    e = jnp.exp(SOFTMAX_TEMPERATURE * (cos - cos[:, 0:1]))
    sm_out[...] = e / jnp.sum(e, axis=1, keepdims=True)

    y = y_ref[...]
    mask = (vals == y).astype(jnp.float32)
    pos = jnp.max(cos * mask, axis=1)
    neg = jnp.max(cos * (1.0 - mask), axis=1)
    hinge = jnp.maximum(neg - pos + MARGIN, 0.0)
    part = jnp.sum(hinge) / B

    @pl.when(i == 0)
    def _():
        acc_s[0] = 0.0

    acc_s[0] += part
    loss_out[...] = jnp.full((1, 1), acc_s[0], jnp.float32)


@jax.jit
def kernel(x, y, keys, values):
    keys_p = jnp.pad(keys, ((0, NPAD - N), (0, 0)))
    vals_p = jnp.pad(values.reshape(-1), (0, NPAD - N)).reshape(NBLK, 128)

    scores, gidx = pl.pallas_call(
        _k1_body,
        grid=(NRB, NCHUNK),
        in_specs=[
            pl.BlockSpec((RB, D), lambda i, j: (i, 0)),
            pl.BlockSpec((CHUNK, D), lambda i, j: (j, 0)),
        ],
        out_specs=[
            pl.BlockSpec((RB, CHUNK), lambda i, j: (i, j)),
            pl.BlockSpec((RB, K), lambda i, j: (i, 0)),
        ],
        out_shape=[
            jax.ShapeDtypeStruct((B, NPAD), jnp.float32),
            jax.ShapeDtypeStruct((B, K), jnp.int32),
        ],
        scratch_shapes=[
            pltpu.VMEM((RB, D), jnp.float32),
            pltpu.VMEM((RB, K), jnp.int32),
        ],
        compiler_params=pltpu.CompilerParams(
            dimension_semantics=("parallel", "arbitrary"),
        ),
    )(x, keys_p)

    scores2d = scores.reshape(B * NBLK, 128)
    bidx = gidx >> 3
    sidx = (bidx + jnp.arange(B, dtype=jnp.int32)[:, None] * NBLK).reshape(1, B * K)
    vidx = bidx.reshape(1, B * K)

    cand_s, cand_v = _sc_gather(scores2d, vals_p, sidx, vidx)

    cand_s = cand_s.reshape(B, K * 128)
    cand_v = cand_v.reshape(B, K * 128)

    y_hat, softmax_score, loss = pl.pallas_call(
        _k3_body,
        grid=(NRB3,),
        in_specs=[
            pl.BlockSpec((RB3, K * 128), lambda i: (i, 0)),
            pl.BlockSpec((RB3, K * 128), lambda i: (i, 0)),
            pl.BlockSpec((RB3, K), lambda i: (i, 0)),
            pl.BlockSpec((RB3, 1), lambda i: (i, 0)),
        ],
        out_specs=[
            pl.BlockSpec((RB3, 1), lambda i: (i, 0)),
            pl.BlockSpec((RB3, K), lambda i: (i, 0)),
            pl.BlockSpec((1, 1), lambda i: (0, 0)),
        ],
        out_shape=[
            jax.ShapeDtypeStruct((B, 1), jnp.int32),
            jax.ShapeDtypeStruct((B, K), jnp.float32),
            jax.ShapeDtypeStruct((1, 1), jnp.float32),
        ],
        scratch_shapes=[pltpu.SMEM((1,), jnp.float32)],
        compiler_params=pltpu.CompilerParams(
            dimension_semantics=("arbitrary",),
        ),
    )(cand_s, cand_v, gidx, y.reshape(B, 1))

    return (y_hat, softmax_score, loss.reshape(()))


def _sc_gather(scores2d, vals_p, flat_idx, gidx_flat):
    nidx = B * K
    win = 128

    vector_mesh = plsc.VectorSubcoreMesh(
        core_axis_name="core", subcore_axis_name="subcore"
    )

    @pl.kernel(
        out_type=[
            jax.ShapeDtypeStruct((nidx, 128), jnp.float32),
            jax.ShapeDtypeStruct((nidx, 128), jnp.int32),
        ],
        mesh=vector_mesh,
    )
    def sc_kernel(s_hbm, v_hbm, i1_hbm, i2_hbm, os_hbm, ov_hbm):
        def body(i1_vmem, i2_vmem, os_vmem, ov_vmem):
            pltpu.sync_copy(s_hbm.at[i1_vmem.at[0]], os_vmem)
            pltpu.sync_copy(v_hbm.at[i2_vmem.at[0]], ov_vmem)

        pltpu.emit_pipeline(
            body,
            grid=(nidx // win,),
            in_specs=[
                pl.BlockSpec((1, win), lambda i: (0, i)),
                pl.BlockSpec((1, win), lambda i: (0, i)),
            ],
            out_specs=[
                pl.BlockSpec((win, 128), lambda i: (i, 0)),
                pl.BlockSpec((win, 128), lambda i: (i, 0)),
            ],
            core_axis_name=("core", "subcore"),
            dimension_semantics=(pltpu.PARALLEL,),
        )(i1_hbm, i2_hbm, os_hbm, ov_hbm)

    return sc_kernel(scores2d, vals_p, flat_idx, gidx_flat)


# speedup vs baseline: 1.4382x; 1.4382x over previous
import functools
import math

import jax
import jax.numpy as jnp
from jax.experimental import pallas as pl
from jax.experimental.pallas import tpu as pltpu
from jax.experimental.pallas import tpu_sc as plsc

B = 1024
D = 64
N = 100000
K = 256
NPAD = 102400
CHUNK = 4096
NCHUNK = NPAD // CHUNK
G = 16
NG = NPAD // G
GPC = CHUNK // G
RB = 128
NRB = B // RB
RB3 = 64
NRB3 = B // RB3
NBLK = NPAD // 128
MARGIN = 0.1
SOFTMAX_TEMPERATURE = max(1.0, math.log(0.2 * K) / 40)

_NEG = -1e30


def _lane_iota(shape, axis):
    return jax.lax.broadcasted_iota(jnp.int32, shape, axis)


def _cmp_exchange(vals, j, want_max):
    key = vals[0]
    up = (_lane_iota(key.shape, key.ndim - 1) & j) == 0
    partners = tuple(
        jnp.where(up, jnp.roll(v, -j, axis=-1), jnp.roll(v, j, axis=-1))
        for v in vals
    )
    pkey = partners[0]
    take_self = jnp.logical_xor(want_max, key < pkey)
    return tuple(
        jnp.where(take_self, v, pv) for v, pv in zip(vals, partners)
    )


def _bitonic_sort(vals, n, desc):
    key = vals[0]
    lane = _lane_iota(key.shape, key.ndim - 1)
    k = 2
    while k <= n:
        blk = (lane & k) == 0 if k < n else jnp.full(key.shape, True)
        xbd = jnp.logical_xor(blk, desc)
        j = k // 2
        while j >= 1:
            up = (lane & j) == 0
            want_max = jnp.logical_xor(up, xbd)
            vals = _cmp_exchange(vals, j, want_max)
            j //= 2
        k *= 2
    return vals


def _bitonic_clean(vals, n, desc):
    key = vals[0]
    lane = _lane_iota(key.shape, key.ndim - 1)
    j = n // 2
    while j >= 1:
        up = (lane & j) == 0
        want_max = jnp.logical_not(jnp.logical_xor(up, desc))
        vals = _cmp_exchange(vals, j, want_max)
        j //= 2
    return vals


def _viota(shape):
    nd = len(shape)
    return _lane_iota(shape, nd - 2) * 128 + _lane_iota(shape, nd - 1)


def _vcmpx(vals, d, want_max):
    key = vals[0]
    if d < 128:
        up = (_lane_iota(key.shape, key.ndim - 1) & d) == 0
        partners = tuple(
            jnp.where(up, jnp.roll(v, -d, axis=-1), jnp.roll(v, d, axis=-1))
            for v in vals
        )
    else:
        d2 = d // 128
        a_sz = key.shape[-2]

        def xorax(v):
            blocks = []
            for base in range(0, a_sz, 2 * d2):
                blocks.append(v[..., base + d2:base + 2 * d2, :])
                blocks.append(v[..., base:base + d2, :])
            return jnp.concatenate(blocks, axis=-2)

        partners = tuple(xorax(v) for v in vals)
    pkey = partners[0]
    take_self = jnp.logical_xor(want_max, key < pkey)
    return tuple(
        jnp.where(take_self, v, pv) for v, pv in zip(vals, partners)
    )


def _vsort(vals, n, desc):
    key = vals[0]
    vio = _viota(key.shape)
    k = 2
    while k <= n:
        blk = (vio & k) == 0 if k < n else jnp.full(key.shape, True)
        xbd = jnp.logical_xor(blk, desc)
        j = k // 2
        while j >= 1:
            up = (vio & j) == 0
            want_max = jnp.logical_xor(up, xbd)
            vals = _vcmpx(vals, j, want_max)
            j //= 2
        k *= 2
    return vals


def _vclean(vals, n, desc):
    key = vals[0]
    vio = _viota(key.shape)
    j = n // 2
    while j >= 1:
        up = (vio & j) == 0
        want_max = jnp.logical_not(jnp.logical_xor(up, desc))
        vals = _vcmpx(vals, j, want_max)
        j //= 2
    return vals


def _k1_body(x_ref, k_ref, s_out, gidx_out, q_s, t_s):
    j = pl.program_id(1)

    @pl.when(j == 0)
    def _():
        xb = x_ref[...]
        nrm = jnp.sqrt(jnp.sum(xb * xb, axis=1, keepdims=True))
        q_s[...] = xb / jnp.maximum(nrm, 1e-12)
        t_s[...] = jnp.full((RB, K), -1, jnp.int32)

    q = q_s[...]
    s = jax.lax.dot_general(
        q, k_ref[...], (((1,), (1,)), ((), ())),
        preferred_element_type=jnp.float32,
    )
    col = _lane_iota((RB, CHUNK), 1) + j * CHUNK
    s = jnp.where(col < N, s, _NEG)
    s_out[...] = s

    mf = s
    for sh in (1, 2, 4, 8):
        mf = jnp.maximum(mf, jnp.roll(mf, -sh, axis=1))
    lsel = _lane_iota((CHUNK, GPC), 0) == _lane_iota((CHUNK, GPC), 1) * G
    psel = lsel.astype(jnp.float32)
    m = jax.lax.dot_general(
        mf, psel, (((1,), (0,)), ((), ())),
        preferred_element_type=jnp.float32,
    )
    u = jnp.clip((m + 1.0) * 131072.0, 0.0, 262143.0).astype(jnp.int32)
    g = _lane_iota((RB, GPC), 1) + j * GPC
    packed = (u << 13) | g

    pm = packed.reshape(RB, 2, 128)
    (ps,) = _vsort((pm,), GPC, False)
    t = t_s[...].reshape(RB, 2, 128)
    z = jnp.maximum(t, ps)
    (t,) = _vclean((z,), K, True)
    t2 = t.reshape(RB, K)
    t_s[...] = t2

    @pl.when(j == NCHUNK - 1)
    def _():
        gidx_out[...] = t2 & 0x1FFF


def _k3_body(sb_ref, vb_ref, gx_ref, y_ref, yhat_out, sm_out, loss_out, acc_s):
    i = pl.program_id(0)
    sb = sb_ref[...].reshape(RB3, K, 128)
    vb = vb_ref[...].reshape(RB3, K, 128)
    sbt = jnp.swapaxes(sb, 1, 2)
    vbt = jnp.swapaxes(vb, 1, 2)
    off = (gx_ref[...] & 7)[:, None, :]

    cs = sbt[:, 0:G, :]
    cv = vbt[:, 0:G, :]
    for o in range(1, 8):
        sel = off == o
        cs = jnp.where(sel, sbt[:, o * G:(o + 1) * G, :], cs)
        cv = jnp.where(sel, vbt[:, o * G:(o + 1) * G, :], cv)

    seg = 16
    desc0 = (_lane_iota((1, seg, 1), 1) < seg // 2)
    cs, cv = _bitonic_sort((cs, cv), K, desc0)
    while seg > 1:
        half = seg // 2
        a_s, b_s = cs[:, :half, :], cs[:, half:, :]
        a_v, b_v = cv[:, :half, :], cv[:, half:, :]
        take_a = a_s >= b_s
        zs = jnp.where(take_a, a_s, b_s)
        zv = jnp.where(take_a, a_v, b_v)
        if half > 1:
            dirn = (_lane_iota((1, half, 1), 1) < half // 2)
        else:
            dirn = True
        cs, cv = _bitonic_clean((zs, zv), K, dirn)
        seg = half
    cos = cs.reshape(RB3, K)
    vals = cv.reshape(RB3, K)

    yhat_out[...] = vals[:, 0:1]

    e = jnp.exp(SOFTMAX_TEMPERATURE * (cos - cos[:, 0:1]))
    sm_out[...] = e / jnp.sum(e, axis=1, keepdims=True)

    y = y_ref[...]
    mask = (vals == y).astype(jnp.float32)
    pos = jnp.max(cos * mask, axis=1)
    neg = jnp.max(cos * (1.0 - mask), axis=1)
    hinge = jnp.maximum(neg - pos + MARGIN, 0.0)
    part = jnp.sum(hinge) / B

    @pl.when(i == 0)
    def _():
        acc_s[0] = 0.0

    acc_s[0] += part
    loss_out[...] = jnp.full((1, 1), acc_s[0], jnp.float32)


@jax.jit
def kernel(x, y, keys, values):
    keys_p = jnp.pad(keys, ((0, NPAD - N), (0, 0)))
    vals_p = jnp.pad(values.reshape(-1), (0, NPAD - N)).reshape(NBLK, 128)

    scores, gidx = pl.pallas_call(
        _k1_body,
        grid=(NRB, NCHUNK),
        in_specs=[
            pl.BlockSpec((RB, D), lambda i, j: (i, 0)),
            pl.BlockSpec((CHUNK, D), lambda i, j: (j, 0)),
        ],
        out_specs=[
            pl.BlockSpec((RB, CHUNK), lambda i, j: (i, j)),
            pl.BlockSpec((RB, K), lambda i, j: (i, 0)),
        ],
        out_shape=[
            jax.ShapeDtypeStruct((B, NPAD), jnp.float32),
            jax.ShapeDtypeStruct((B, K), jnp.int32),
        ],
        scratch_shapes=[
            pltpu.VMEM((RB, D), jnp.float32),
            pltpu.VMEM((RB, K), jnp.int32),
        ],
        compiler_params=pltpu.CompilerParams(
            dimension_semantics=("parallel", "arbitrary"),
        ),
    )(x, keys_p)

    scores2d = scores.reshape(B * NBLK, 128)
    bidx = gidx >> 3
    sidx = (bidx + jnp.arange(B, dtype=jnp.int32)[:, None] * NBLK).reshape(1, B * K)
    vidx = bidx.reshape(1, B * K)

    cand_s, cand_v = _sc_gather(scores2d, vals_p, sidx, vidx)

    cand_s = cand_s.reshape(B, K * 128)
    cand_v = cand_v.reshape(B, K * 128)

    y_hat, softmax_score, loss = pl.pallas_call(
        _k3_body,
        grid=(NRB3,),
        in_specs=[
            pl.BlockSpec((RB3, K * 128), lambda i: (i, 0)),
            pl.BlockSpec((RB3, K * 128), lambda i: (i, 0)),
            pl.BlockSpec((RB3, K), lambda i: (i, 0)),
            pl.BlockSpec((RB3, 1), lambda i: (i, 0)),
        ],
        out_specs=[
            pl.BlockSpec((RB3, 1), lambda i: (i, 0)),
            pl.BlockSpec((RB3, K), lambda i: (i, 0)),
            pl.BlockSpec((1, 1), lambda i: (0, 0)),
        ],
        out_shape=[
            jax.ShapeDtypeStruct((B, 1), jnp.int32),
            jax.ShapeDtypeStruct((B, K), jnp.float32),
            jax.ShapeDtypeStruct((1, 1), jnp.float32),
        ],
        scratch_shapes=[pltpu.SMEM((1,), jnp.float32)],
        compiler_params=pltpu.CompilerParams(
            dimension_semantics=("arbitrary",),
        ),
    )(cand_s, cand_v, gidx, y.reshape(B, 1))

    return (y_hat, softmax_score, loss.reshape(()))


def _sc_gather(scores2d, vals_p, flat_idx, gidx_flat):
    nidx = B * K
    win = 128

    vector_mesh = plsc.VectorSubcoreMesh(
        core_axis_name="core", subcore_axis_name="subcore"
    )

    @pl.kernel(
        out_type=[
            jax.ShapeDtypeStruct((nidx, 128), jnp.float32),
            jax.ShapeDtypeStruct((nidx, 128), jnp.int32),
        ],
        mesh=vector_mesh,
    )
    def sc_kernel(s_hbm, v_hbm, i1_hbm, i2_hbm, os_hbm, ov_hbm):
        def body(i1_vmem, i2_vmem, os_vmem, ov_vmem):
            pltpu.sync_copy(s_hbm.at[i1_vmem.at[0]], os_vmem)
            pltpu.sync_copy(v_hbm.at[i2_vmem.at[0]], ov_vmem)

        pltpu.emit_pipeline(
            body,
            grid=(nidx // win,),
            in_specs=[
                pl.BlockSpec((1, win), lambda i: (0, i)),
                pl.BlockSpec((1, win), lambda i: (0, i)),
            ],
            out_specs=[
                pl.BlockSpec((win, 128), lambda i: (i, 0)),
                pl.BlockSpec((win, 128), lambda i: (i, 0)),
            ],
            core_axis_name=("core", "subcore"),
            dimension_semantics=(pltpu.PARALLEL,),
        )(i1_hbm, i2_hbm, os_hbm, ov_hbm)

    return sc_kernel(scores2d, vals_p, flat_idx, gidx_flat)


# speedup vs baseline: 1.7877x; 1.2430x over previous
import functools
import math

import jax
import jax.numpy as jnp
from jax.experimental import pallas as pl
from jax.experimental.pallas import tpu as pltpu
from jax.experimental.pallas import tpu_sc as plsc

B = 1024
D = 64
N = 100000
K = 256
NPAD = 106496
CHUNK = 8192
NCHUNK = NPAD // CHUNK
HCH = CHUNK // 2
G = 16
NG = NPAD // G
GPC = CHUNK // G
RB = 128
NRB = B // RB
RB3 = 64
NRB3 = B // RB3
NBLK = NPAD // 128
MARGIN = 0.1
SOFTMAX_TEMPERATURE = max(1.0, math.log(0.2 * K) / 40)

_NEG = -1e30


def _lane_iota(shape, axis):
    return jax.lax.broadcasted_iota(jnp.int32, shape, axis)


def _cmp_exchange(vals, j, want_max):
    key = vals[0]
    up = (_lane_iota(key.shape, key.ndim - 1) & j) == 0
    partners = tuple(
        jnp.where(up, jnp.roll(v, -j, axis=-1), jnp.roll(v, j, axis=-1))
        for v in vals
    )
    pkey = partners[0]
    take_self = jnp.logical_xor(want_max, key < pkey)
    return tuple(
        jnp.where(take_self, v, pv) for v, pv in zip(vals, partners)
    )


def _bitonic_sort(vals, n, desc):
    key = vals[0]
    lane = _lane_iota(key.shape, key.ndim - 1)
    k = 2
    while k <= n:
        blk = (lane & k) == 0 if k < n else jnp.full(key.shape, True)
        xbd = jnp.logical_xor(blk, desc)
        j = k // 2
        while j >= 1:
            up = (lane & j) == 0
            want_max = jnp.logical_xor(up, xbd)
            vals = _cmp_exchange(vals, j, want_max)
            j //= 2
        k *= 2
    return vals


def _bitonic_clean(vals, n, desc):
    key = vals[0]
    lane = _lane_iota(key.shape, key.ndim - 1)
    j = n // 2
    while j >= 1:
        up = (lane & j) == 0
        want_max = jnp.logical_not(jnp.logical_xor(up, desc))
        vals = _cmp_exchange(vals, j, want_max)
        j //= 2
    return vals


def _viota(shape):
    nd = len(shape)
    return _lane_iota(shape, nd - 2) * 128 + _lane_iota(shape, nd - 1)


def _vcmpx(vals, d, want_max):
    key = vals[0]
    if d < 128:
        up = (_lane_iota(key.shape, key.ndim - 1) & d) == 0
        partners = tuple(
            jnp.where(up, jnp.roll(v, -d, axis=-1), jnp.roll(v, d, axis=-1))
            for v in vals
        )
    else:
        d2 = d // 128
        a_sz = key.shape[-2]

        def xorax(v):
            blocks = []
            for base in range(0, a_sz, 2 * d2):
                blocks.append(v[..., base + d2:base + 2 * d2, :])
                blocks.append(v[..., base:base + d2, :])
            return jnp.concatenate(blocks, axis=-2)

        partners = tuple(xorax(v) for v in vals)
    pkey = partners[0]
    take_self = jnp.logical_xor(want_max, key < pkey)
    return tuple(
        jnp.where(take_self, v, pv) for v, pv in zip(vals, partners)
    )


def _vsort(vals, n, desc):
    key = vals[0]
    vio = _viota(key.shape)
    k = 2
    while k <= n:
        blk = (vio & k) == 0 if k < n else jnp.full(key.shape, True)
        xbd = jnp.logical_xor(blk, desc)
        j = k // 2
        while j >= 1:
            up = (vio & j) == 0
            want_max = jnp.logical_xor(up, xbd)
            vals = _vcmpx(vals, j, want_max)
            j //= 2
        k *= 2
    return vals


def _vclean(vals, n, desc):
    key = vals[0]
    vio = _viota(key.shape)
    j = n // 2
    while j >= 1:
        up = (vio & j) == 0
        want_max = jnp.logical_not(jnp.logical_xor(up, desc))
        vals = _vcmpx(vals, j, want_max)
        j //= 2
    return vals


def _k1_body(x_ref, k_ref, s_out, gidx_out, q_s, t_s):
    j = pl.program_id(1)

    @pl.when(j == 0)
    def _():
        xb = x_ref[...]
        nrm = jnp.sqrt(jnp.sum(xb * xb, axis=1, keepdims=True))
        q_s[...] = xb / jnp.maximum(nrm, 1e-12)
        t_s[...] = jnp.full((RB, K), -1, jnp.int32)

    q = q_s[...]
    s = jax.lax.dot_general(
        q, k_ref[...], (((1,), (1,)), ((), ())),
        preferred_element_type=jnp.float32,
    )
    col = _lane_iota((RB, CHUNK), 1) + j * CHUNK
    s = jnp.where(col < N, s, _NEG)
    s_out[...] = s

    mf = s
    for sh in (1, 2, 4, 8):
        mf = jnp.maximum(mf, jnp.roll(mf, -sh, axis=1))
    lsel = _lane_iota((HCH, HCH // G), 0) == _lane_iota((HCH, HCH // G), 1) * G
    psel = lsel.astype(jnp.float32)
    m1 = jax.lax.dot_general(
        mf[:, :HCH], psel, (((1,), (0,)), ((), ())),
        preferred_element_type=jnp.float32,
    )
    m2 = jax.lax.dot_general(
        mf[:, HCH:], psel, (((1,), (0,)), ((), ())),
        preferred_element_type=jnp.float32,
    )
    m = jnp.concatenate([m1, m2], axis=1)
    u = jnp.clip((m + 1.0) * 131072.0, 0.0, 262143.0).astype(jnp.int32)
    g = _lane_iota((RB, GPC), 1) + j * GPC
    packed = (u << 13) | g

    pm = packed.reshape(RB, 4, 128)
    (ps,) = _vsort((pm,), GPC, False)
    t = t_s[...].reshape(RB, 2, 128)
    z = jnp.maximum(t, ps[:, 2:4, :])
    (t,) = _vclean((z,), K, True)
    t2 = t.reshape(RB, K)
    t_s[...] = t2

    @pl.when(j == NCHUNK - 1)
    def _():
        gidx_out[...] = t2 & 0x1FFF


def _k3_body(sb_ref, vb_ref, gx_ref, y_ref, yhat_out, sm_out, loss_out, acc_s):
    i = pl.program_id(0)
    sb = sb_ref[...].reshape(RB3, K, 128)
    vb = vb_ref[...].reshape(RB3, K, 128)
    sbt = jnp.swapaxes(sb, 1, 2)
    vbt = jnp.swapaxes(vb, 1, 2)
    off = (gx_ref[...] & 7)[:, None, :]

    cs = sbt[:, 0:G, :]
    cv = vbt[:, 0:G, :]
    for o in range(1, 8):
        sel = off == o
        cs = jnp.where(sel, sbt[:, o * G:(o + 1) * G, :], cs)
        cv = jnp.where(sel, vbt[:, o * G:(o + 1) * G, :], cv)

    seg = 16
    desc0 = (_lane_iota((1, seg, 1), 1) < seg // 2)
    cs, cv = _bitonic_sort((cs, cv), K, desc0)
    while seg > 1:
        half = seg // 2
        a_s, b_s = cs[:, :half, :], cs[:, half:, :]
        a_v, b_v = cv[:, :half, :], cv[:, half:, :]
        take_a = a_s >= b_s
        zs = jnp.where(take_a, a_s, b_s)
        zv = jnp.where(take_a, a_v, b_v)
        if half > 1:
            dirn = (_lane_iota((1, half, 1), 1) < half // 2)
        else:
            dirn = True
        cs, cv = _bitonic_clean((zs, zv), K, dirn)
        seg = half
    cos = cs.reshape(RB3, K)
    vals = cv.reshape(RB3, K)

    yhat_out[...] = vals[:, 0:1]

    e = jnp.exp(SOFTMAX_TEMPERATURE * (cos - cos[:, 0:1]))
    sm_out[...] = e / jnp.sum(e, axis=1, keepdims=True)

    y = y_ref[...]
    mask = (vals == y).astype(jnp.float32)
    pos = jnp.max(cos * mask, axis=1)
    neg = jnp.max(cos * (1.0 - mask), axis=1)
    hinge = jnp.maximum(neg - pos + MARGIN, 0.0)
    part = jnp.sum(hinge) / B

    @pl.when(i == 0)
    def _():
        acc_s[0] = 0.0

    acc_s[0] += part
    loss_out[...] = jnp.full((1, 1), acc_s[0], jnp.float32)


@jax.jit
def kernel(x, y, keys, values):
    keys_p = jnp.pad(keys, ((0, NPAD - N), (0, 0)))
    vals_p = jnp.pad(values.reshape(-1), (0, NPAD - N)).reshape(NBLK, 128)

    scores, gidx = pl.pallas_call(
        _k1_body,
        grid=(NRB, NCHUNK),
        in_specs=[
            pl.BlockSpec((RB, D), lambda i, j: (i, 0)),
            pl.BlockSpec((CHUNK, D), lambda i, j: (j, 0)),
        ],
        out_specs=[
            pl.BlockSpec((RB, CHUNK), lambda i, j: (i, j)),
            pl.BlockSpec((RB, K), lambda i, j: (i, 0)),
        ],
        out_shape=[
            jax.ShapeDtypeStruct((B, NPAD), jnp.float32),
            jax.ShapeDtypeStruct((B, K), jnp.int32),
        ],
        scratch_shapes=[
            pltpu.VMEM((RB, D), jnp.float32),
            pltpu.VMEM((RB, K), jnp.int32),
        ],
        compiler_params=pltpu.CompilerParams(
            dimension_semantics=("parallel", "arbitrary"),
        ),
    )(x, keys_p)

    scores2d = scores.reshape(B * NBLK, 128)
    bidx = gidx >> 3
    sidx = (bidx + jnp.arange(B, dtype=jnp.int32)[:, None] * NBLK).reshape(1, B * K)
    vidx = bidx.reshape(1, B * K)

    cand_s, cand_v = _sc_gather(scores2d, vals_p, sidx, vidx)

    cand_s = cand_s.reshape(B, K * 128)
    cand_v = cand_v.reshape(B, K * 128)

    y_hat, softmax_score, loss = pl.pallas_call(
        _k3_body,
        grid=(NRB3,),
        in_specs=[
            pl.BlockSpec((RB3, K * 128), lambda i: (i, 0)),
            pl.BlockSpec((RB3, K * 128), lambda i: (i, 0)),
            pl.BlockSpec((RB3, K), lambda i: (i, 0)),
            pl.BlockSpec((RB3, 1), lambda i: (i, 0)),
        ],
        out_specs=[
            pl.BlockSpec((RB3, 1), lambda i: (i, 0)),
            pl.BlockSpec((RB3, K), lambda i: (i, 0)),
            pl.BlockSpec((1, 1), lambda i: (0, 0)),
        ],
        out_shape=[
            jax.ShapeDtypeStruct((B, 1), jnp.int32),
            jax.ShapeDtypeStruct((B, K), jnp.float32),
            jax.ShapeDtypeStruct((1, 1), jnp.float32),
        ],
        scratch_shapes=[pltpu.SMEM((1,), jnp.float32)],
        compiler_params=pltpu.CompilerParams(
            dimension_semantics=("arbitrary",),
        ),
    )(cand_s, cand_v, gidx, y.reshape(B, 1))

    return (y_hat, softmax_score, loss.reshape(()))


def _sc_gather(scores2d, vals_p, flat_idx, gidx_flat):
    nidx = B * K
    win = 128

    vector_mesh = plsc.VectorSubcoreMesh(
        core_axis_name="core", subcore_axis_name="subcore"
    )

    @pl.kernel(
        out_type=[
            jax.ShapeDtypeStruct((nidx, 128), jnp.float32),
            jax.ShapeDtypeStruct((nidx, 128), jnp.int32),
        ],
        mesh=vector_mesh,
    )
    def sc_kernel(s_hbm, v_hbm, i1_hbm, i2_hbm, os_hbm, ov_hbm):
        def body(i1_vmem, i2_vmem, os_vmem, ov_vmem):
            pltpu.sync_copy(s_hbm.at[i1_vmem.at[0]], os_vmem)
            pltpu.sync_copy(v_hbm.at[i2_vmem.at[0]], ov_vmem)

        pltpu.emit_pipeline(
            body,
            grid=(nidx // win,),
            in_specs=[
                pl.BlockSpec((1, win), lambda i: (0, i)),
                pl.BlockSpec((1, win), lambda i: (0, i)),
            ],
            out_specs=[
                pl.BlockSpec((win, 128), lambda i: (i, 0)),
                pl.BlockSpec((win, 128), lambda i: (i, 0)),
            ],
            core_axis_name=("core", "subcore"),
            dimension_semantics=(pltpu.PARALLEL,),
        )(i1_hbm, i2_hbm, os_hbm, ov_hbm)

    return sc_kernel(scores2d, vals_p, flat_idx, gidx_flat)


# speedup vs baseline: 2.0233x; 1.1318x over previous
import functools
import math

import jax
import jax.numpy as jnp
from jax.experimental import pallas as pl
from jax.experimental.pallas import tpu as pltpu
from jax.experimental.pallas import tpu_sc as plsc

B = 1024
D = 64
N = 100000
K = 256
NPAD = 114688
CHUNK = 16384
NCHUNK = NPAD // CHUNK
HCH = 4096
G = 16
NG = NPAD // G
GPC = CHUNK // G
RB = 128
NRB = B // RB
RB3 = 64
NRB3 = B // RB3
NBLK = NPAD // 128
MARGIN = 0.1
SOFTMAX_TEMPERATURE = max(1.0, math.log(0.2 * K) / 40)

_NEG = -1e30


def _lane_iota(shape, axis):
    return jax.lax.broadcasted_iota(jnp.int32, shape, axis)


def _cmp_exchange(vals, j, want_max):
    key = vals[0]
    up = (_lane_iota(key.shape, key.ndim - 1) & j) == 0
    partners = tuple(
        jnp.where(up, jnp.roll(v, -j, axis=-1), jnp.roll(v, j, axis=-1))
        for v in vals
    )
    pkey = partners[0]
    take_self = jnp.logical_xor(want_max, key < pkey)
    return tuple(
        jnp.where(take_self, v, pv) for v, pv in zip(vals, partners)
    )


def _bitonic_sort(vals, n, desc):
    key = vals[0]
    lane = _lane_iota(key.shape, key.ndim - 1)
    k = 2
    while k <= n:
        blk = (lane & k) == 0 if k < n else jnp.full(key.shape, True)
        xbd = jnp.logical_xor(blk, desc)
        j = k // 2
        while j >= 1:
            up = (lane & j) == 0
            want_max = jnp.logical_xor(up, xbd)
            vals = _cmp_exchange(vals, j, want_max)
            j //= 2
        k *= 2
    return vals


def _bitonic_clean(vals, n, desc):
    key = vals[0]
    lane = _lane_iota(key.shape, key.ndim - 1)
    j = n // 2
    while j >= 1:
        up = (lane & j) == 0
        want_max = jnp.logical_not(jnp.logical_xor(up, desc))
        vals = _cmp_exchange(vals, j, want_max)
        j //= 2
    return vals


def _viota(shape):
    nd = len(shape)
    return _lane_iota(shape, nd - 2) * 128 + _lane_iota(shape, nd - 1)


def _vcmpx(vals, d, want_max):
    key = vals[0]
    if d < 128:
        up = (_lane_iota(key.shape, key.ndim - 1) & d) == 0
        partners = tuple(
            jnp.where(up, jnp.roll(v, -d, axis=-1), jnp.roll(v, d, axis=-1))
            for v in vals
        )
    else:
        d2 = d // 128
        a_sz = key.shape[-2]

        def xorax(v):
            blocks = []
            for base in range(0, a_sz, 2 * d2):
                blocks.append(v[..., base + d2:base + 2 * d2, :])
                blocks.append(v[..., base:base + d2, :])
            return jnp.concatenate(blocks, axis=-2)

        partners = tuple(xorax(v) for v in vals)
    pkey = partners[0]
    take_self = jnp.logical_xor(want_max, key < pkey)
    return tuple(
        jnp.where(take_self, v, pv) for v, pv in zip(vals, partners)
    )


def _vsort(vals, n, desc):
    key = vals[0]
    vio = _viota(key.shape)
    k = 2
    while k <= n:
        blk = (vio & k) == 0 if k < n else jnp.full(key.shape, True)
        xbd = jnp.logical_xor(blk, desc)
        j = k // 2
        while j >= 1:
            up = (vio & j) == 0
            want_max = jnp.logical_xor(up, xbd)
            vals = _vcmpx(vals, j, want_max)
            j //= 2
        k *= 2
    return vals


def _vclean(vals, n, desc):
    key = vals[0]
    vio = _viota(key.shape)
    j = n // 2
    while j >= 1:
        up = (vio & j) == 0
        want_max = jnp.logical_not(jnp.logical_xor(up, desc))
        vals = _vcmpx(vals, j, want_max)
        j //= 2
    return vals


def _k1_body(x_ref, k_ref, s_out, gidx_out, q_s, t_s):
    j = pl.program_id(1)

    @pl.when(j == 0)
    def _():
        xb = x_ref[...]
        nrm = jnp.sqrt(jnp.sum(xb * xb, axis=1, keepdims=True))
        q_s[...] = xb / jnp.maximum(nrm, 1e-12)
        t_s[...] = jnp.full((RB, K), -1, jnp.int32)

    q = q_s[...]
    s = jax.lax.dot_general(
        q, k_ref[...], (((1,), (1,)), ((), ())),
        preferred_element_type=jnp.float32,
    )
    col = _lane_iota((RB, CHUNK), 1) + j * CHUNK
    s = jnp.where(col < N, s, _NEG)
    s_out[...] = s

    mf = s
    for sh in (1, 2, 4, 8):
        mf = jnp.maximum(mf, jnp.roll(mf, -sh, axis=1))
    lsel = _lane_iota((HCH, HCH // G), 0) == _lane_iota((HCH, HCH // G), 1) * G
    psel = lsel.astype(jnp.float32)
    parts = [
        jax.lax.dot_general(
            mf[:, h * HCH:(h + 1) * HCH], psel, (((1,), (0,)), ((), ())),
            preferred_element_type=jnp.float32,
        )
        for h in range(CHUNK // HCH)
    ]
    m = jnp.concatenate(parts, axis=1)
    u = jnp.clip((m + 1.0) * 131072.0, 0.0, 262143.0).astype(jnp.int32)
    g = _lane_iota((RB, GPC), 1) + j * GPC
    packed = (u << 13) | g

    pm = packed.reshape(RB, 8, 128)
    (ps,) = _vsort((pm,), GPC, False)
    t = t_s[...].reshape(RB, 2, 128)
    z = jnp.maximum(t, ps[:, 6:8, :])
    (t,) = _vclean((z,), K, True)
    t2 = t.reshape(RB, K)
    t_s[...] = t2

    @pl.when(j == NCHUNK - 1)
    def _():
        gidx_out[...] = t2 & 0x1FFF


def _k3_body(sb_ref, vb_ref, gx_ref, y_ref, yhat_out, sm_out, loss_out, acc_s):
    i = pl.program_id(0)
    sb = sb_ref[...].reshape(RB3, K, 128)
    vb = vb_ref[...].reshape(RB3, K, 128)
    sbt = jnp.swapaxes(sb, 1, 2)
    vbt = jnp.swapaxes(vb, 1, 2)
    off = (gx_ref[...] & 7)[:, None, :]

    cs = sbt[:, 0:G, :]
    cv = vbt[:, 0:G, :]
    for o in range(1, 8):
        sel = off == o
        cs = jnp.where(sel, sbt[:, o * G:(o + 1) * G, :], cs)
        cv = jnp.where(sel, vbt[:, o * G:(o + 1) * G, :], cv)

    seg = 16
    desc0 = (_lane_iota((1, seg, 1), 1) < seg // 2)
    cs, cv = _bitonic_sort((cs, cv), K, desc0)
    while seg > 1:
        half = seg // 2
        a_s, b_s = cs[:, :half, :], cs[:, half:, :]
        a_v, b_v = cv[:, :half, :], cv[:, half:, :]
        take_a = a_s >= b_s
        zs = jnp.where(take_a, a_s, b_s)
        zv = jnp.where(take_a, a_v, b_v)
        if half > 1:
            dirn = (_lane_iota((1, half, 1), 1) < half // 2)
        else:
            dirn = True
        cs, cv = _bitonic_clean((zs, zv), K, dirn)
        seg = half
    cos = cs.reshape(RB3, K)
    vals = cv.reshape(RB3, K)

    yhat_out[...] = vals[:, 0:1]

    e = jnp.exp(SOFTMAX_TEMPERATURE * (cos - cos[:, 0:1]))
    sm_out[...] = e / jnp.sum(e, axis=1, keepdims=True)

    y = y_ref[...]
    mask = (vals == y).astype(jnp.float32)
    pos = jnp.max(cos * mask, axis=1)
    neg = jnp.max(cos * (1.0 - mask), axis=1)
    hinge = jnp.maximum(neg - pos + MARGIN, 0.0)
    part = jnp.sum(hinge) / B

    @pl.when(i == 0)
    def _():
        acc_s[0] = 0.0

    acc_s[0] += part
    loss_out[...] = jnp.full((1, 1), acc_s[0], jnp.float32)


@jax.jit
def kernel(x, y, keys, values):
    keys_p = jnp.pad(keys, ((0, NPAD - N), (0, 0)))
    vals_p = jnp.pad(values.reshape(-1), (0, NPAD - N)).reshape(NBLK, 128)

    scores, gidx = pl.pallas_call(
        _k1_body,
        grid=(NRB, NCHUNK),
        in_specs=[
            pl.BlockSpec((RB, D), lambda i, j: (i, 0)),
            pl.BlockSpec((CHUNK, D), lambda i, j: (j, 0)),
        ],
        out_specs=[
            pl.BlockSpec((RB, CHUNK), lambda i, j: (i, j)),
            pl.BlockSpec((RB, K), lambda i, j: (i, 0)),
        ],
        out_shape=[
            jax.ShapeDtypeStruct((B, NPAD), jnp.float32),
            jax.ShapeDtypeStruct((B, K), jnp.int32),
        ],
        scratch_shapes=[
            pltpu.VMEM((RB, D), jnp.float32),
            pltpu.VMEM((RB, K), jnp.int32),
        ],
        compiler_params=pltpu.CompilerParams(
            dimension_semantics=("parallel", "arbitrary"),
        ),
    )(x, keys_p)

    scores2d = scores.reshape(B * NBLK, 128)
    bidx = gidx >> 3
    sidx = (bidx + jnp.arange(B, dtype=jnp.int32)[:, None] * NBLK).reshape(1, B * K)
    vidx = bidx.reshape(1, B * K)

    cand_s, cand_v = _sc_gather(scores2d, vals_p, sidx, vidx)

    cand_s = cand_s.reshape(B, K * 128)
    cand_v = cand_v.reshape(B, K * 128)

    y_hat, softmax_score, loss = pl.pallas_call(
        _k3_body,
        grid=(NRB3,),
        in_specs=[
            pl.BlockSpec((RB3, K * 128), lambda i: (i, 0)),
            pl.BlockSpec((RB3, K * 128), lambda i: (i, 0)),
            pl.BlockSpec((RB3, K), lambda i: (i, 0)),
            pl.BlockSpec((RB3, 1), lambda i: (i, 0)),
        ],
        out_specs=[
            pl.BlockSpec((RB3, 1), lambda i: (i, 0)),
            pl.BlockSpec((RB3, K), lambda i: (i, 0)),
            pl.BlockSpec((1, 1), lambda i: (0, 0)),
        ],
        out_shape=[
            jax.ShapeDtypeStruct((B, 1), jnp.int32),
            jax.ShapeDtypeStruct((B, K), jnp.float32),
            jax.ShapeDtypeStruct((1, 1), jnp.float32),
        ],
        scratch_shapes=[pltpu.SMEM((1,), jnp.float32)],
        compiler_params=pltpu.CompilerParams(
            dimension_semantics=("arbitrary",),
        ),
    )(cand_s, cand_v, gidx, y.reshape(B, 1))

    return (y_hat, softmax_score, loss.reshape(()))


def _sc_gather(scores2d, vals_p, flat_idx, gidx_flat):
    nidx = B * K
    win = 128

    vector_mesh = plsc.VectorSubcoreMesh(
        core_axis_name="core", subcore_axis_name="subcore"
    )

    @pl.kernel(
        out_type=[
            jax.ShapeDtypeStruct((nidx, 128), jnp.float32),
            jax.ShapeDtypeStruct((nidx, 128), jnp.int32),
        ],
        mesh=vector_mesh,
    )
    def sc_kernel(s_hbm, v_hbm, i1_hbm, i2_hbm, os_hbm, ov_hbm):
        def body(i1_vmem, i2_vmem, os_vmem, ov_vmem):
            pltpu.sync_copy(s_hbm.at[i1_vmem.at[0]], os_vmem)
            pltpu.sync_copy(v_hbm.at[i2_vmem.at[0]], ov_vmem)

        pltpu.emit_pipeline(
            body,
            grid=(nidx // win,),
            in_specs=[
                pl.BlockSpec((1, win), lambda i: (0, i)),
                pl.BlockSpec((1, win), lambda i: (0, i)),
            ],
            out_specs=[
                pl.BlockSpec((win, 128), lambda i: (i, 0)),
                pl.BlockSpec((win, 128), lambda i: (i, 0)),
            ],
            core_axis_name=("core", "subcore"),
            dimension_semantics=(pltpu.PARALLEL,),
        )(i1_hbm, i2_hbm, os_hbm, ov_hbm)

    return sc_kernel(scores2d, vals_p, flat_idx, gidx_flat)
